# Initial kernel scaffold; baseline (speedup 1.0000x reference)
import jax
import jax.numpy as jnp
from jax.experimental import pallas as pl


def kernel(input):
    raise NotImplementedError("write your pallas kernel here")


# speedup vs baseline: 16.3981x; 16.3981x over previous
import functools
import math

import jax
import jax.numpy as jnp
from jax import lax
from jax.experimental import pallas as pl
from jax.experimental.pallas import tpu as pltpu
from jax.experimental.pallas import tpu_sc as plsc

D = 128
HALF = 64
TAB = 208
STARTPOS = 1025.0
LOG_BASE = math.log(10000.0) / (HALF - 1)


def _table_body(o_ref):
    ri = lax.broadcasted_iota(jnp.int32, (TAB, D), 0)
    c = lax.broadcasted_iota(jnp.int32, (TAB, D), 1)
    r = ri.astype(jnp.float32)
    k = jnp.where(c < HALF, c, c - HALF).astype(jnp.float32)
    freq = jnp.exp(k * (-LOG_BASE))
    ang = (STARTPOS + r) * freq
    val = jnp.where(c < HALF, jnp.sin(ang), jnp.cos(ang))
    dead = (ri == 1) | (ri > 201)
    o_ref[...] = jnp.where(dead, 0.0, val)


def _build_table():
    return pl.pallas_call(
        _table_body,
        out_shape=jax.ShapeDtypeStruct((TAB, D), jnp.float32),
    )()


def _sc_kernel(seq_len, rows_per_w, nc, table_hbm, inp_hbm, out_hbm,
               tab_sh, inp_v, pos_v, row_a, row_b, sem_a, sem_b):
    n_chunks = rows_per_w * seq_len // D
    cid = lax.axis_index("c")
    sid = lax.axis_index("s")
    wid = sid * nc + cid
    row0 = wid * rows_per_w
    obase = row0 * seq_len

    @pl.when(sid == 0)
    def _():
        pltpu.sync_copy(table_hbm, tab_sh)
    plsc.subcore_barrier()

    pltpu.sync_copy(inp_hbm.at[pl.ds(obase, rows_per_w * seq_len)], inp_v)

    lane = lax.iota(jnp.int32, 16)

    def seq_body(j, carry):
        s_vec, done, off = carry
        tok = plsc.load_gather(inp_v, [off])
        s_vec = jnp.where(tok == 4, j, s_vec)
        done = jnp.where(tok == 1, 1, done)
        pos = jnp.where(done > 0, 1, (j + 2) - s_vec)
        plsc.store_scatter(pos_v, [off], pos)
        return s_vec, done, off + 1

    def grp_body(g, _):
        off = (lane + g * 16) * seq_len
        zero = jnp.zeros((16,), jnp.int32)
        lax.fori_loop(0, seq_len, seq_body, (zero, zero, off))
        return 0

    lax.fori_loop(0, rows_per_w // 16, grp_body, 0)

    def gather(k, buf, sem):
        idx = pos_v.at[pl.ds(k * D, D)]
        pltpu.async_copy(tab_sh.at[idx], buf, sem)

    gather(0, row_a, sem_a)

    def chunk_body(k, _):
        even = lax.rem(k, 2) == 0
        buf, sem = row_a, sem_a

        @pl.when(k + 1 < n_chunks)
        def _():
            @pl.when(even)
            def _():
                gather(k + 1, row_b, sem_b)

            @pl.when(jnp.logical_not(even))
            def _():
                gather(k + 1, row_a, sem_a)

        @pl.when(even)
        def _():
            idx = pos_v.at[pl.ds(k * D, D)]
            pltpu.make_async_copy(tab_sh.at[idx], row_a, sem_a).wait()
            pltpu.sync_copy(row_a, out_hbm.at[pl.ds(obase + k * D, D)])

        @pl.when(jnp.logical_not(even))
        def _():
            idx = pos_v.at[pl.ds(k * D, D)]
            pltpu.make_async_copy(tab_sh.at[idx], row_b, sem_b).wait()
            pltpu.sync_copy(row_b, out_hbm.at[pl.ds(obase + k * D, D)])

        return 0

    lax.fori_loop(0, n_chunks, chunk_body, 0)


def _lookup(table, inp):
    bsz, seq_len = inp.shape
    info = plsc.get_sparse_core_info()
    nw = info.num_cores * info.num_subcores
    rows_per_w = bsz // nw
    n_chunks = rows_per_w * seq_len // D
    mesh = plsc.VectorSubcoreMesh(core_axis_name="c", subcore_axis_name="s")
    body = functools.partial(_sc_kernel, seq_len, rows_per_w,
                             info.num_cores)
    return pl.kernel(
        body,
        out_type=jax.ShapeDtypeStruct((bsz * seq_len, D), jnp.float32),
        mesh=mesh,
        scratch_types=[
            pltpu.VMEM_SHARED((TAB, D), jnp.float32),
            pltpu.VMEM((rows_per_w * seq_len,), jnp.int32),
            pltpu.VMEM((rows_per_w * seq_len,), jnp.int32),
            pltpu.VMEM((D, D), jnp.float32),
            pltpu.VMEM((D, D), jnp.float32),
            pltpu.SemaphoreType.DMA,
            pltpu.SemaphoreType.DMA,
        ],
        compiler_params=pltpu.CompilerParams(
            needs_layout_passes=False,
            use_tc_tiling_on_sc=False,
        ),
    )(table, inp.reshape(-1))


def kernel(input):
    bsz, seq_len = input.shape
    table = _build_table()
    out = _lookup(table, input)
    return out.reshape(bsz, seq_len, D)
